# Initial kernel scaffold; baseline (speedup 1.0000x reference)
import jax
import jax.numpy as jnp
from jax.experimental import pallas as pl


def kernel(legal_ids, logits):
    raise NotImplementedError("write your pallas kernel here")


# speedup vs baseline: 36.5253x; 36.5253x over previous
import functools

import jax
import jax.numpy as jnp
import numpy as np
from jax import lax
from jax.experimental import pallas as pl
from jax.experimental.pallas import tpu as pltpu
from jax.experimental.pallas import tpu_sc as plsc

_NUM_MOVES = 1968
_NEG = -1e30


def _threefry2x32(x0, x1):
    k0 = jnp.uint32(0)
    k1 = jnp.uint32(1)
    ks = [k0, k1, k0 ^ k1 ^ jnp.uint32(0x1BD11BDA)]
    rot_a = [13, 15, 26, 6]
    rot_b = [17, 29, 16, 24]

    def rotl(x, r):
        return (x << jnp.uint32(r)) | (x >> jnp.uint32(32 - r))

    x0 = x0 + ks[0]
    x1 = x1 + ks[1]
    for i, rots in enumerate([rot_a, rot_b, rot_a, rot_b, rot_a]):
        for r in rots:
            x0 = x0 + x1
            x1 = rotl(x1, r)
            x1 = x0 ^ x1
        x0 = x0 + ks[(i + 1) % 3]
        x1 = x1 + ks[(i + 2) % 3] + jnp.uint32(i + 1)
    return x0, x1


def _gumbel_from_flat_idx(flat_idx):
    i = flat_idx.astype(jnp.uint32)
    z0, z1 = _threefry2x32(jnp.zeros_like(i), i)
    bits = z0 ^ z1
    f = lax.bitcast_convert_type(
        (bits >> jnp.uint32(9)) | jnp.uint32(0x3F800000), jnp.float32
    ) - jnp.float32(1.0)
    span = np.float32(1.0) - np.float32(1e-10)
    u = jnp.maximum(jnp.float32(1e-10), f * span + jnp.float32(1e-10))
    return -jnp.log(-jnp.log(u))


def _sc_gather_dedup(logits, flat_ids):
    n = flat_ids.shape[0]
    info = plsc.get_sparse_core_info()
    nw = info.num_cores * info.num_subcores
    per = n // nw
    rows_per = per // 64
    mesh = plsc.VectorSubcoreMesh(core_axis_name="c", subcore_axis_name="s")

    @functools.partial(
        pl.kernel,
        mesh=mesh,
        compiler_params=pltpu.CompilerParams(needs_layout_passes=False),
        out_type=[
            jax.ShapeDtypeStruct((n,), jnp.float32),
            jax.ShapeDtypeStruct((n,), jnp.float32),
        ],
        scratch_types=[
            pltpu.VMEM((_NUM_MOVES,), jnp.float32),
            pltpu.VMEM((_NUM_MOVES,), jnp.int32),
            pltpu.VMEM((per,), jnp.int32),
            pltpu.VMEM((per,), jnp.float32),
            pltpu.VMEM((per,), jnp.float32),
        ],
    )
    def sc_kernel(logits_hbm, ids_hbm, g_hbm, rep_hbm,
                  table_v, slot_v, idx_v, g_v, rep_v):
        wid = lax.axis_index("s") * info.num_cores + lax.axis_index("c")
        base = wid * per
        pltpu.sync_copy(logits_hbm, table_v)
        pltpu.sync_copy(ids_hbm.at[pl.ds(base, per)], idx_v)
        lane = lax.iota(jnp.int32, 16)

        def row_body(r, carry):
            rb = pl.multiple_of(r * 64, 64)
            idxs = []
            for k in range(4):
                sl = pl.ds(rb + k * 16, 16)
                idx = idx_v[sl]
                idxs.append(idx)
                g_v[sl] = plsc.load_gather(table_v, [idx])
                plsc.store_scatter(slot_v, [idx], lane + jnp.int32(k * 16))
            for k in range(4):
                sl = pl.ds(rb + k * 16, 16)
                winner = plsc.load_gather(slot_v, [idxs[k]])
                rep_v[sl] = jnp.where(
                    winner == lane + jnp.int32(k * 16),
                    jnp.float32(1.0), jnp.float32(0.0))
            return carry

        lax.fori_loop(0, rows_per, row_body, 0)
        pltpu.sync_copy(g_v, g_hbm.at[pl.ds(base, per)])
        pltpu.sync_copy(rep_v, rep_hbm.at[pl.ds(base, per)])

    return sc_kernel(logits, flat_ids)


def _tc_policy(legal_ids, g, rep, block_rows):
    b, l = legal_ids.shape
    grid = (b // block_rows,)

    def body(ids_ref, g_ref, rep_ref, sample_ref, logp_ref):
        ids = ids_ref[...]
        gv = g_ref[...]
        repv = rep_ref[...]
        rows = pl.program_id(0) * block_rows + lax.broadcasted_iota(
            jnp.int32, (block_rows, l), 0)
        gum = _gumbel_from_flat_idx(rows * jnp.int32(_NUM_MOVES) + ids)
        m = jnp.max(gv, axis=1, keepdims=True)
        e = jnp.exp(gv - m)
        z = jnp.sum(repv * e, axis=1, keepdims=True)
        logp = jnp.log(e / z + jnp.float32(1e-30))
        cand = logp + gum
        maxv = jnp.max(cand, axis=1, keepdims=True)
        samp = jnp.min(jnp.where(cand == maxv, ids, jnp.int32(2**30)), axis=1)
        sample_ref[...] = samp
        logp_ref[...] = jnp.min(
            jnp.where(ids == samp[:, None], logp, jnp.float32(3e38)), axis=1)

    sample, logp = pl.pallas_call(
        body,
        grid=grid,
        in_specs=[
            pl.BlockSpec((block_rows, l), lambda i: (i, 0)),
            pl.BlockSpec((block_rows, l), lambda i: (i, 0)),
            pl.BlockSpec((block_rows, l), lambda i: (i, 0)),
        ],
        out_specs=[
            pl.BlockSpec((block_rows,), lambda i: (i,)),
            pl.BlockSpec((block_rows,), lambda i: (i,)),
        ],
        out_shape=[
            jax.ShapeDtypeStruct((b,), jnp.int32),
            jax.ShapeDtypeStruct((b,), jnp.float32),
        ],
    )(legal_ids, g, rep)
    return sample, logp


def kernel(legal_ids, logits):
    b, l = legal_ids.shape
    g_flat, rep_flat = _sc_gather_dedup(logits, legal_ids.reshape(-1))
    sample, logp = _tc_policy(
        legal_ids, g_flat.reshape(b, l), rep_flat.reshape(b, l), 256)
    return sample, logp[:, None]


# speedup vs baseline: 49.7911x; 1.3632x over previous
import functools

import jax
import jax.numpy as jnp
import numpy as np
from jax import lax
from jax.experimental import pallas as pl
from jax.experimental.pallas import tpu as pltpu
from jax.experimental.pallas import tpu_sc as plsc

_NUM_MOVES = 1968
_NEG = jnp.float32(-1e30)


def _threefry2x32(x0, x1):
    k0 = jnp.uint32(0)
    k1 = jnp.uint32(1)
    ks = [k0, k1, k0 ^ k1 ^ jnp.uint32(0x1BD11BDA)]
    rot_a = [13, 15, 26, 6]
    rot_b = [17, 29, 16, 24]

    def rotl(x, r):
        return (x << jnp.uint32(r)) | (x >> jnp.uint32(32 - r))

    x0 = x0 + ks[0]
    x1 = x1 + ks[1]
    for i, rots in enumerate([rot_a, rot_b, rot_a, rot_b, rot_a]):
        for r in rots:
            x0 = x0 + x1
            x1 = rotl(x1, r)
            x1 = x0 ^ x1
        x0 = x0 + ks[(i + 1) % 3]
        x1 = x1 + ks[(i + 2) % 3] + jnp.uint32(i + 1)
    return x0, x1


def _gumbel_from_flat_idx(flat_idx):
    i = flat_idx.astype(jnp.uint32)
    z0, z1 = _threefry2x32(jnp.zeros_like(i), i)
    bits = z0 ^ z1
    f = lax.bitcast_convert_type(
        (bits >> jnp.uint32(9)) | jnp.uint32(0x3F800000), jnp.float32
    ) - jnp.float32(1.0)
    span = np.float32(1.0) - np.float32(1e-10)
    u = jnp.maximum(jnp.float32(1e-10), f * span + jnp.float32(1e-10))
    return -jnp.log(-jnp.log(u))


def _sc_gather_mask(logits, flat_ids):
    n = flat_ids.shape[0]
    info = plsc.get_sparse_core_info()
    nw = info.num_cores * info.num_subcores
    per = n // nw
    rows_per = per // 64
    mesh = plsc.VectorSubcoreMesh(core_axis_name="c", subcore_axis_name="s")

    @functools.partial(
        pl.kernel,
        mesh=mesh,
        compiler_params=pltpu.CompilerParams(needs_layout_passes=False),
        out_type=jax.ShapeDtypeStruct((n,), jnp.float32),
        scratch_types=[
            pltpu.VMEM((_NUM_MOVES,), jnp.float32),
            pltpu.VMEM((_NUM_MOVES,), jnp.int32),
            pltpu.VMEM((per,), jnp.int32),
            pltpu.VMEM((per,), jnp.float32),
        ],
    )
    def sc_kernel(logits_hbm, ids_hbm, gm_hbm, table_v, slot_v, idx_v, gm_v):
        wid = lax.axis_index("s") * info.num_cores + lax.axis_index("c")
        base = wid * per
        pltpu.sync_copy(logits_hbm, table_v)
        pltpu.sync_copy(ids_hbm.at[pl.ds(base, per)], idx_v)
        lane = lax.iota(jnp.int32, 16)

        def row_body(r, carry):
            rb = pl.multiple_of(r * 64, 64)
            idxs = []
            gs = []
            for k in range(4):
                sl = pl.ds(rb + k * 16, 16)
                idx = idx_v[sl]
                idxs.append(idx)
                gs.append(plsc.load_gather(table_v, [idx]))
                plsc.store_scatter(slot_v, [idx], lane + jnp.int32(k * 16))
            for k in range(4):
                sl = pl.ds(rb + k * 16, 16)
                winner = plsc.load_gather(slot_v, [idxs[k]])
                gm_v[sl] = jnp.where(
                    winner == lane + jnp.int32(k * 16), gs[k], _NEG)
            return carry

        lax.fori_loop(0, rows_per, row_body, 0)
        pltpu.sync_copy(gm_v, gm_hbm.at[pl.ds(base, per)])

    return sc_kernel(logits, flat_ids)


def _tc_gumbel(ids_wide, block_rows):
    n, w = ids_wide.shape
    grid = (n // block_rows,)

    def body(ids_ref, gum_ref):
        ids = ids_ref[...]
        p = (pl.program_id(0) * block_rows) * w + lax.broadcasted_iota(
            jnp.int32, (block_rows, w), 0) * w + lax.broadcasted_iota(
            jnp.int32, (block_rows, w), 1)
        row = lax.shift_right_logical(p, 6)
        gum_ref[...] = _gumbel_from_flat_idx(row * jnp.int32(_NUM_MOVES) + ids)

    return pl.pallas_call(
        body,
        grid=grid,
        in_specs=[pl.BlockSpec((block_rows, w), lambda i: (i, 0))],
        out_specs=pl.BlockSpec((block_rows, w), lambda i: (i, 0)),
        out_shape=jax.ShapeDtypeStruct((n, w), jnp.float32),
    )(ids_wide)


def _tc_combine(legal_ids, gm, gum, block_rows):
    b, l = legal_ids.shape
    grid = (b // block_rows,)

    def body(ids_ref, gm_ref, gum_ref, sample_ref, logp_ref):
        idsf = ids_ref[...].astype(jnp.float32)
        gv = gm_ref[...]
        m = jnp.max(gv, axis=1, keepdims=True)
        e = jnp.exp(gv - m)
        z = jnp.sum(e, axis=1, keepdims=True)
        logp = jnp.log(e / z + jnp.float32(1e-30))
        cand = logp + gum_ref[...]
        maxv = jnp.max(cand, axis=1, keepdims=True)
        sampf = jnp.min(
            jnp.where(cand == maxv, idsf, jnp.float32(3e38)), axis=1)
        sample_ref[...] = sampf.astype(jnp.int32)
        logp_ref[...] = jnp.max(
            jnp.where(idsf == sampf[:, None], logp, jnp.float32(-3e38)), axis=1)

    return pl.pallas_call(
        body,
        grid=grid,
        in_specs=[
            pl.BlockSpec((block_rows, l), lambda i: (i, 0)),
            pl.BlockSpec((block_rows, l), lambda i: (i, 0)),
            pl.BlockSpec((block_rows, l), lambda i: (i, 0)),
        ],
        out_specs=[
            pl.BlockSpec((block_rows,), lambda i: (i,)),
            pl.BlockSpec((block_rows,), lambda i: (i,)),
        ],
        out_shape=[
            jax.ShapeDtypeStruct((b,), jnp.int32),
            jax.ShapeDtypeStruct((b,), jnp.float32),
        ],
    )(legal_ids, gm, gum)


def kernel(legal_ids, logits):
    b, l = legal_ids.shape
    flat_ids = legal_ids.reshape(-1)
    gm_flat = _sc_gather_mask(logits, flat_ids)
    gum_wide = _tc_gumbel(flat_ids.reshape(b * l // 128, 128), 512)
    sample, logp = _tc_combine(
        legal_ids, gm_flat.reshape(b, l), gum_wide.reshape(b, l), 512)
    return sample, logp[:, None]


# speedup vs baseline: 54.2642x; 1.0898x over previous
import functools

import jax
import jax.numpy as jnp
import numpy as np
from jax import lax
from jax.experimental import pallas as pl
from jax.experimental.pallas import tpu as pltpu
from jax.experimental.pallas import tpu_sc as plsc

_NUM_MOVES = 1968
_NEG = np.float32(-1e30)


def _threefry2x32(x0, x1):
    k0 = jnp.uint32(0)
    k1 = jnp.uint32(1)
    ks = [k0, k1, k0 ^ k1 ^ jnp.uint32(0x1BD11BDA)]
    rot_a = [13, 15, 26, 6]
    rot_b = [17, 29, 16, 24]

    def rotl(x, r):
        return (x << jnp.uint32(r)) | (x >> jnp.uint32(32 - r))

    x0 = x0 + ks[0]
    x1 = x1 + ks[1]
    for i, rots in enumerate([rot_a, rot_b, rot_a, rot_b, rot_a]):
        for r in rots:
            x0 = x0 + x1
            x1 = rotl(x1, r)
            x1 = x0 ^ x1
        x0 = x0 + ks[(i + 1) % 3]
        x1 = x1 + ks[(i + 2) % 3] + jnp.uint32(i + 1)
    return x0, x1


def _gumbel_from_flat_idx(flat_idx):
    i = flat_idx.astype(jnp.uint32)
    z0, z1 = _threefry2x32(jnp.zeros_like(i), i)
    bits = z0 ^ z1
    f = lax.bitcast_convert_type(
        (bits >> jnp.uint32(9)) | jnp.uint32(0x3F800000), jnp.float32
    ) - jnp.float32(1.0)
    span = np.float32(1.0) - np.float32(1e-10)
    u = jnp.maximum(jnp.float32(1e-10), f * span + jnp.float32(1e-10))
    return -jnp.log(-jnp.log(u))


def _sc_gather_mask(logits, flat_ids):
    n = flat_ids.shape[0]
    info = plsc.get_sparse_core_info()
    nw = info.num_cores * info.num_subcores
    per = n // nw
    rows_per = per // 64
    mesh = plsc.VectorSubcoreMesh(core_axis_name="c", subcore_axis_name="s")

    @functools.partial(
        pl.kernel,
        mesh=mesh,
        compiler_params=pltpu.CompilerParams(needs_layout_passes=False),
        out_type=jax.ShapeDtypeStruct((n,), jnp.float32),
        scratch_types=[
            pltpu.VMEM((_NUM_MOVES,), jnp.float32),
            pltpu.VMEM((_NUM_MOVES,), jnp.int32),
            pltpu.VMEM((per,), jnp.int32),
            pltpu.VMEM((per,), jnp.float32),
        ],
    )
    def sc_kernel(logits_hbm, ids_hbm, gm_hbm, table_v, slot_v, idx_v, gm_v):
        wid = lax.axis_index("s") * info.num_cores + lax.axis_index("c")
        base = wid * per
        pltpu.sync_copy(logits_hbm, table_v)
        pltpu.sync_copy(ids_hbm.at[pl.ds(base, per)], idx_v)
        lane = lax.iota(jnp.int32, 16)

        def row_body(r, carry):
            rb = pl.multiple_of(r * 64, 64)
            idxs = []
            gs = []
            for k in range(4):
                sl = pl.ds(rb + k * 16, 16)
                idx = idx_v[sl]
                idxs.append(idx)
                gs.append(plsc.load_gather(table_v, [idx]))
                plsc.store_scatter(slot_v, [idx], lane + jnp.int32(k * 16))
            for k in range(4):
                sl = pl.ds(rb + k * 16, 16)
                winner = plsc.load_gather(slot_v, [idxs[k]])
                gm_v[sl] = jnp.where(
                    winner == lane + jnp.int32(k * 16), gs[k], _NEG)
            return carry

        lax.fori_loop(0, rows_per, row_body, 0)
        pltpu.sync_copy(gm_v, gm_hbm.at[pl.ds(base, per)])

    return sc_kernel(logits, flat_ids)


def _tc_gumbel(ids_wide, block_rows):
    n, w = ids_wide.shape
    grid = (n // block_rows,)

    def body(ids_ref, gum_ref):
        ids = ids_ref[...]
        p = (pl.program_id(0) * block_rows) * w + lax.broadcasted_iota(
            jnp.int32, (block_rows, w), 0) * w + lax.broadcasted_iota(
            jnp.int32, (block_rows, w), 1)
        row = lax.shift_right_logical(p, 6)
        gum_ref[...] = _gumbel_from_flat_idx(row * jnp.int32(_NUM_MOVES) + ids)

    return pl.pallas_call(
        body,
        grid=grid,
        in_specs=[pl.BlockSpec((block_rows, w), lambda i: (i, 0))],
        out_specs=pl.BlockSpec((block_rows, w), lambda i: (i, 0)),
        out_shape=jax.ShapeDtypeStruct((n, w), jnp.float32),
    )(ids_wide)


def _tc_combine(ids_wide, gm_wide, gum_wide, b, l, wide_block):
    nw = b * l // 128
    grid = (nw // wide_block,)
    w = wide_block

    def body(ids_ref, gm_ref, gum_ref, se_ref, so_ref, le_ref, lo_ref):
        idsf = ids_ref[...].astype(jnp.float32)
        gv = gm_ref[...]

        def seg(x, red):
            a = red(x[:, :64], axis=1, keepdims=True)
            c = red(x[:, 64:], axis=1, keepdims=True)
            return jnp.concatenate(
                [jnp.broadcast_to(a, (w, 64)), jnp.broadcast_to(c, (w, 64))],
                axis=1)

        m = seg(gv, jnp.max)
        e = jnp.exp(gv - m)
        z = seg(e, jnp.sum)
        logp = jnp.log(e / z + jnp.float32(1e-30))
        cand = logp + gum_ref[...]
        maxv = seg(cand, jnp.max)
        wids = jnp.where(cand == maxv, idsf, jnp.float32(3e38))
        sa = jnp.min(wids[:, :64], axis=1)
        sc = jnp.min(wids[:, 64:], axis=1)
        se_ref[...] = sa.astype(jnp.int32)
        so_ref[...] = sc.astype(jnp.int32)
        samp = jnp.concatenate(
            [jnp.broadcast_to(sa[:, None], (w, 64)),
             jnp.broadcast_to(sc[:, None], (w, 64))], axis=1)
        lp = jnp.where(idsf == samp, logp, jnp.float32(-3e38))
        le_ref[...] = jnp.max(lp[:, :64], axis=1)
        lo_ref[...] = jnp.max(lp[:, 64:], axis=1)

    return pl.pallas_call(
        body,
        grid=grid,
        in_specs=[
            pl.BlockSpec((w, 128), lambda i: (i, 0)),
            pl.BlockSpec((w, 128), lambda i: (i, 0)),
            pl.BlockSpec((w, 128), lambda i: (i, 0)),
        ],
        out_specs=[
            pl.BlockSpec((w,), lambda i: (i,)),
            pl.BlockSpec((w,), lambda i: (i,)),
            pl.BlockSpec((w,), lambda i: (i,)),
            pl.BlockSpec((w,), lambda i: (i,)),
        ],
        out_shape=[
            jax.ShapeDtypeStruct((nw,), jnp.int32),
            jax.ShapeDtypeStruct((nw,), jnp.int32),
            jax.ShapeDtypeStruct((nw,), jnp.float32),
            jax.ShapeDtypeStruct((nw,), jnp.float32),
        ],
    )(ids_wide, gm_wide, gum_wide)


def kernel(legal_ids, logits):
    b, l = legal_ids.shape
    nw = b * l // 128
    flat_ids = legal_ids.reshape(-1)
    ids_wide = flat_ids.reshape(nw, 128)
    gm_flat = _sc_gather_mask(logits, flat_ids)
    gum_wide = _tc_gumbel(ids_wide, 512)
    se, so, le, lo = _tc_combine(
        ids_wide, gm_flat.reshape(nw, 128), gum_wide, b, l, 512)
    sample = jnp.stack([se, so], axis=1).reshape(b)
    logp = jnp.stack([le, lo], axis=1).reshape(b, 1)
    return sample, logp


# speedup vs baseline: 60.0605x; 1.1068x over previous
import functools

import jax
import jax.numpy as jnp
import numpy as np
from jax import lax
from jax.experimental import pallas as pl
from jax.experimental.pallas import tpu as pltpu
from jax.experimental.pallas import tpu_sc as plsc

_NUM_MOVES = 1968
_NEG = np.float32(-1e30)


def _threefry2x32(x0, x1):
    k0 = jnp.uint32(0)
    k1 = jnp.uint32(1)
    ks = [k0, k1, k0 ^ k1 ^ jnp.uint32(0x1BD11BDA)]
    rot_a = [13, 15, 26, 6]
    rot_b = [17, 29, 16, 24]

    def rotl(x, r):
        return (x << jnp.uint32(r)) | (x >> jnp.uint32(32 - r))

    x0 = x0 + ks[0]
    x1 = x1 + ks[1]
    for i, rots in enumerate([rot_a, rot_b, rot_a, rot_b, rot_a]):
        for r in rots:
            x0 = x0 + x1
            x1 = rotl(x1, r)
            x1 = x0 ^ x1
        x0 = x0 + ks[(i + 1) % 3]
        x1 = x1 + ks[(i + 2) % 3] + jnp.uint32(i + 1)
    return x0, x1


def _gumbel_from_flat_idx(flat_idx):
    i = flat_idx.astype(jnp.uint32)
    z0, z1 = _threefry2x32(jnp.zeros_like(i), i)
    bits = z0 ^ z1
    f = lax.bitcast_convert_type(
        (bits >> jnp.uint32(9)) | jnp.uint32(0x3F800000), jnp.float32
    ) - jnp.float32(1.0)
    span = np.float32(1.0) - np.float32(1e-10)
    u = jnp.maximum(jnp.float32(1e-10), f * span + jnp.float32(1e-10))
    return -jnp.log(-jnp.log(u))


def _sc_gather_mask(logits, flat_ids):
    n = flat_ids.shape[0]
    info = plsc.get_sparse_core_info()
    nw = info.num_cores * info.num_subcores
    per = n // nw
    rows_per = per // 64
    mesh = plsc.VectorSubcoreMesh(core_axis_name="c", subcore_axis_name="s")

    @functools.partial(
        pl.kernel,
        mesh=mesh,
        compiler_params=pltpu.CompilerParams(needs_layout_passes=False),
        out_type=jax.ShapeDtypeStruct((n,), jnp.float32),
        scratch_types=[
            pltpu.VMEM((_NUM_MOVES,), jnp.float32),
            pltpu.VMEM((_NUM_MOVES,), jnp.int32),
            pltpu.VMEM((_NUM_MOVES,), jnp.int32),
            pltpu.VMEM((_NUM_MOVES,), jnp.int32),
            pltpu.VMEM((_NUM_MOVES,), jnp.int32),
            pltpu.VMEM((per,), jnp.int32),
            pltpu.VMEM((per,), jnp.float32),
        ],
    )
    def sc_kernel(logits_hbm, ids_hbm, gm_hbm, table_v, slot_v, slot_v2,
                  slot_v3, slot_v4, idx_v, gm_v):
        wid = lax.axis_index("s") * info.num_cores + lax.axis_index("c")
        base = wid * per
        pltpu.sync_copy(logits_hbm, table_v)
        pltpu.sync_copy(ids_hbm.at[pl.ds(base, per)], idx_v)
        lane = lax.iota(jnp.int32, 16)

        def row_body(r4, carry):
            rb = pl.multiple_of(r4 * 256, 256)
            for j, slot_t in enumerate((slot_v, slot_v2, slot_v3, slot_v4)):
                idxs = []
                gs = []
                for k in range(4):
                    sl = pl.ds(rb + j * 64 + k * 16, 16)
                    idx = idx_v[sl]
                    idxs.append(idx)
                    gs.append(plsc.load_gather(table_v, [idx]))
                    plsc.store_scatter(slot_t, [idx], lane + jnp.int32(k * 16))
                for k in range(4):
                    sl = pl.ds(rb + j * 64 + k * 16, 16)
                    winner = plsc.load_gather(slot_t, [idxs[k]])
                    gm_v[sl] = jnp.where(
                        winner == lane + jnp.int32(k * 16), gs[k], _NEG)
            return carry

        lax.fori_loop(0, rows_per // 4, row_body, 0)
        pltpu.sync_copy(gm_v, gm_hbm.at[pl.ds(base, per)])

    return sc_kernel(logits, flat_ids)


def _tc_gumbel(ids_wide, half_b, block_rows):
    n, w = ids_wide.shape
    grid = (n // block_rows,)

    def body(ids_ref, gum_ref):
        ids = ids_ref[...]
        wrow = pl.program_id(0) * block_rows + lax.broadcasted_iota(
            jnp.int32, (block_rows, w), 0)
        lanes = lax.broadcasted_iota(jnp.int32, (block_rows, w), 1)
        row = wrow + jnp.where(lanes >= 64, jnp.int32(half_b), jnp.int32(0))
        gum_ref[...] = _gumbel_from_flat_idx(row * jnp.int32(_NUM_MOVES) + ids)

    return pl.pallas_call(
        body,
        grid=grid,
        in_specs=[pl.BlockSpec((block_rows, w), lambda i: (i, 0))],
        out_specs=pl.BlockSpec((block_rows, w), lambda i: (i, 0)),
        out_shape=jax.ShapeDtypeStruct((n, w), jnp.float32),
    )(ids_wide)


def _tc_combine(ids_wide, gm_wide, gum_wide, b, l, wide_block):
    nw = b * l // 128
    grid = (nw // wide_block,)
    w = wide_block

    def body(ids_ref, gm_ref, gum_ref, se_ref, so_ref, le_ref, lo_ref):
        idsf = ids_ref[...].astype(jnp.float32)
        gv = gm_ref[...]

        def seg(x, red):
            a = red(x[:, :64], axis=1, keepdims=True)
            c = red(x[:, 64:], axis=1, keepdims=True)
            return jnp.concatenate(
                [jnp.broadcast_to(a, (w, 64)), jnp.broadcast_to(c, (w, 64))],
                axis=1)

        m = seg(gv, jnp.max)
        e = jnp.exp(gv - m)
        z = seg(e, jnp.sum)
        logp = jnp.log(e / z + jnp.float32(1e-30))
        cand = logp + gum_ref[...]
        maxv = seg(cand, jnp.max)
        wids = jnp.where(cand == maxv, idsf, jnp.float32(3e38))
        sa = jnp.min(wids[:, :64], axis=1)
        sc = jnp.min(wids[:, 64:], axis=1)
        se_ref[...] = sa.astype(jnp.int32)
        so_ref[...] = sc.astype(jnp.int32)
        samp = jnp.concatenate(
            [jnp.broadcast_to(sa[:, None], (w, 64)),
             jnp.broadcast_to(sc[:, None], (w, 64))], axis=1)
        lp = jnp.where(idsf == samp, logp, jnp.float32(-3e38))
        le_ref[...] = jnp.max(lp[:, :64], axis=1)
        lo_ref[...] = jnp.max(lp[:, 64:], axis=1)

    return pl.pallas_call(
        body,
        grid=grid,
        in_specs=[
            pl.BlockSpec((w, 128), lambda i: (i, 0)),
            pl.BlockSpec((w, 128), lambda i: (i, 0)),
            pl.BlockSpec((w, 128), lambda i: (i, 0)),
        ],
        out_specs=[
            pl.BlockSpec((w,), lambda i: (i,)),
            pl.BlockSpec((w,), lambda i: (i,)),
            pl.BlockSpec((w,), lambda i: (i,)),
            pl.BlockSpec((w,), lambda i: (i,)),
        ],
        out_shape=[
            jax.ShapeDtypeStruct((nw,), jnp.int32),
            jax.ShapeDtypeStruct((nw,), jnp.int32),
            jax.ShapeDtypeStruct((nw,), jnp.float32),
            jax.ShapeDtypeStruct((nw,), jnp.float32),
        ],
    )(ids_wide, gm_wide, gum_wide)


def kernel(legal_ids, logits):
    b, l = legal_ids.shape
    nw = b * l // 128
    ids_wide = jnp.concatenate([legal_ids[: b // 2], legal_ids[b // 2:]],
                               axis=1)
    flat_ids = ids_wide.reshape(-1)
    gm_flat = _sc_gather_mask(logits, flat_ids)
    gum_wide = _tc_gumbel(ids_wide, b // 2, 512)
    s_lo, s_hi, l_lo, l_hi = _tc_combine(
        ids_wide, gm_flat.reshape(nw, 128), gum_wide, b, l, 512)
    sample = jnp.concatenate([s_lo, s_hi])
    logp = jnp.concatenate([l_lo, l_hi]).reshape(b, 1)
    return sample, logp
